# baseline (speedup 1.0000x reference)
import functools

import jax
import jax.numpy as jnp
from jax import lax
from jax.experimental import pallas as pl
from jax.experimental.pallas import tpu as pltpu
from jax.experimental.pallas import tpu_sc as plsc

_N = 10000
_E = 320000
_D = 128
_H = 32
_NG = 64

_NC = 2
_NS = 16
_NW = _NC * _NS
_CH = 128
_K = 80
_EP = _NW * _K * _CH
_NR = 10112
_RT = _NR // _NS
_TS = _NR // _NS
_NT = 4 * _NR
_DW = 16


def _sc_mesh():
    return plsc.VectorSubcoreMesh(
        core_axis_name="c", subcore_axis_name="s",
        num_cores=_NC, num_subcores=_NS)


@functools.cache
def _deg_kernel():
    @functools.partial(
        pl.kernel,
        out_type=jax.ShapeDtypeStruct((_NC, _NR, _DW), jnp.float32),
        mesh=_sc_mesh(),
        compiler_params=pltpu.CompilerParams(use_tc_tiling_on_sc=False),
        scratch_types=[
            pltpu.VMEM((_K, _CH), jnp.int32),
            pltpu.VMEM((_CH, _DW), jnp.float32),
            pltpu.VMEM_SHARED((_NR, _DW), jnp.float32),
        ],
    )
    def deg(dst3, zeros16, ones16, out, dst_v, ones_v, acc):
        c = lax.axis_index("c")
        s = lax.axis_index("s")
        wid = s * _NC + c
        pltpu.sync_copy(dst3.at[wid], dst_v)
        pltpu.sync_copy(ones16, ones_v)
        pltpu.sync_copy(zeros16.at[pl.ds(s * _RT, _RT)],
                        acc.at[pl.ds(s * _RT, _RT)])
        plsc.subcore_barrier()

        def body(j, carry):
            pltpu.sync_copy(ones_v, acc.at[dst_v.at[j]], add=True)
            return carry

        lax.fori_loop(0, _K, body, 0)
        plsc.subcore_barrier()
        pltpu.sync_copy(acc.at[pl.ds(s * _RT, _RT)],
                        out.at[c, pl.ds(s * _RT, _RT)])

    return deg


@functools.cache
def _conv_kernel():
    nbuf = 8
    hb = nbuf // 2

    @functools.partial(
        pl.kernel,
        out_type=jax.ShapeDtypeStruct((_NC, _NR, _H), jnp.float32),
        mesh=_sc_mesh(),
        compiler_params=pltpu.CompilerParams(use_tc_tiling_on_sc=False),
        scratch_types=[
            pltpu.VMEM((_K, _CH), jnp.int32),
            pltpu.VMEM((_K, _CH), jnp.int32),
            pltpu.VMEM((nbuf, _CH, _H), jnp.float32),
            pltpu.VMEM_SHARED((_NR, _H), jnp.float32),
            pltpu.VMEM_SHARED((_NR, _H), jnp.float32),
            pltpu.SemaphoreType.DMA,
            pltpu.SemaphoreType.DMA,
        ],
    )
    def conv(table, src3, dst3, zerosH, out,
             src_v, dst_v, rows_v, table_sp, acc, sem_a, sem_b):
        c = lax.axis_index("c")
        s = lax.axis_index("s")
        wid = s * _NC + c
        pltpu.sync_copy(src3.at[wid], src_v)
        pltpu.sync_copy(dst3.at[wid], dst_v)
        pltpu.sync_copy(table.at[pl.ds(s * _TS, _TS)],
                        table_sp.at[pl.ds(s * _TS, _TS)])
        pltpu.sync_copy(zerosH.at[pl.ds(s * _RT, _RT)],
                        acc.at[pl.ds(s * _RT, _RT)])
        plsc.subcore_barrier()

        ng = _K // nbuf

        def fire(cbase, bufbase, sem):
            for b in range(hb):
                pltpu.async_copy(table_sp.at[src_v.at[cbase + b]],
                                 rows_v.at[bufbase + b], sem)

        def drain_scatter(cbase, bufbase, sem):
            for b in range(hb):
                pltpu.make_async_copy(table_sp.at[src_v.at[cbase + b]],
                                      rows_v.at[bufbase + b], sem).wait()
            for b in range(hb):
                pltpu.sync_copy(rows_v.at[bufbase + b],
                                acc.at[dst_v.at[cbase + b]], add=True)

        fire(0, 0, sem_a)

        def group(g, carry):
            base = g * nbuf
            fire(base + hb, hb, sem_b)
            drain_scatter(base, 0, sem_a)

            @pl.when(g < ng - 1)
            def _():
                fire(base + nbuf, 0, sem_a)

            drain_scatter(base + hb, hb, sem_b)
            return carry

        lax.fori_loop(0, ng, group, 0)
        plsc.subcore_barrier()
        pltpu.sync_copy(acc.at[pl.ds(s * _RT, _RT)],
                        out.at[c, pl.ds(s * _RT, _RT)])

    return conv


@functools.cache
def _mpnn_kernel():
    nbuf = 16
    hb = nbuf // 2

    @functools.partial(
        pl.kernel,
        out_type=jax.ShapeDtypeStruct((_NC, _NR, _H), jnp.float32),
        mesh=_sc_mesh(),
        compiler_params=pltpu.CompilerParams(use_tc_tiling_on_sc=False,
                                             needs_layout_passes=False),
        scratch_types=[
            pltpu.VMEM((_K, _CH), jnp.int32),
            pltpu.VMEM((_K, _CH), jnp.int32),
            pltpu.VMEM((_K, _CH), jnp.int32),
            pltpu.VMEM((nbuf, _CH), jnp.int32),
            pltpu.VMEM((nbuf, _CH, _H // 2), jnp.int32),
            pltpu.VMEM((_CH, _H), jnp.float32),
            pltpu.VMEM_SHARED((_NR, _H), jnp.float32),
            pltpu.SemaphoreType.DMA,
            pltpu.SemaphoreType.DMA,
        ],
    )
    def mpnn(table4, src3, dst3, attr3, zerosH, out,
             src_v, dst_v, attr_v, ridx_v, rows_v, rowf_v, acc,
             sem_a, sem_b):
        c = lax.axis_index("c")
        s = lax.axis_index("s")
        wid = s * _NC + c
        pltpu.sync_copy(src3.at[wid], src_v)
        pltpu.sync_copy(dst3.at[wid], dst_v)
        pltpu.sync_copy(attr3.at[wid], attr_v)
        pltpu.sync_copy(zerosH.at[pl.ds(s * _RT, _RT)],
                        acc.at[pl.ds(s * _RT, _RT)])
        plsc.subcore_barrier()

        ng = _K // nbuf

        def fire(cbase, bufbase, sem):
            for b in range(hb):
                j = cbase + b
                bb = bufbase + b
                for q in range(_CH // 16):
                    sv = src_v[j, pl.ds(q * 16, 16)]
                    av = attr_v[j, pl.ds(q * 16, 16)]
                    ridx_v[bb, pl.ds(q * 16, 16)] = sv * 4 + av
                pltpu.async_copy(table4.at[ridx_v.at[bb]],
                                 rows_v.at[bb], sem)

        def drain_scatter(cbase, bufbase, sem):
            for b in range(hb):
                bb = bufbase + b
                pltpu.make_async_copy(table4.at[ridx_v.at[bb]],
                                      rows_v.at[bb], sem).wait()

                def expand(r, carry):
                    for u in range(4):
                        x = rows_v[bb, r * 4 + u]
                        lo = plsc.bitcast(lax.shift_left(x, 16),
                                          jnp.float32)
                        hi = plsc.bitcast(
                            jnp.bitwise_and(x, jnp.int32(-65536)),
                            jnp.float32)
                        rowf_v[r * 4 + u, pl.ds(0, 16)] = lo
                        rowf_v[r * 4 + u, pl.ds(16, 16)] = hi
                    return carry

                lax.fori_loop(0, _CH // 4, expand, 0)
                pltpu.sync_copy(rowf_v, acc.at[dst_v.at[cbase + b]],
                                add=True)

        fire(0, 0, sem_a)

        def group(g, carry):
            base = g * nbuf
            fire(base + hb, hb, sem_b)
            drain_scatter(base, 0, sem_a)

            @pl.when(g < ng - 1)
            def _():
                fire(base + nbuf, 0, sem_a)

            drain_scatter(base + hb, hb, sem_b)
            return carry

        lax.fori_loop(0, ng, group, 0)
        plsc.subcore_barrier()
        pltpu.sync_copy(acc.at[pl.ds(s * _RT, _RT)],
                        out.at[c, pl.ds(s * _RT, _RT)])

    return mpnn


def _pad_rows(a, rows):
    return jnp.concatenate(
        [a, jnp.zeros((rows - a.shape[0], a.shape[1]), a.dtype)], axis=0)


def _mm_body(x_ref, w_ref, o_ref):
    o_ref[...] = jnp.dot(x_ref[...], w_ref[...],
                         preferred_element_type=jnp.float32)


def _tcb_body(degp_ref, xw1_ref, dis_ref, cnt_ref, xws_ref):
    dp = degp_ref[...]
    cnt = dp[0, :_N, 0:1] + dp[1, :_N, 0:1]
    dis = lax.rsqrt(cnt + 1.0)
    dis_ref[...] = dis
    cnt_ref[...] = cnt
    xws_ref[...] = _pad_rows(xw1_ref[...] * dis, _NR)


def _tcc1_body(sp_ref, xw_ref, dis_ref, b_ref, w2_ref, xw2_ref, xws2_ref):
    spv = sp_ref[...]
    ssum = spv[0, :_N, :] + spv[1, :_N, :]
    d = dis_ref[...]
    h = jnp.maximum(d * ssum + (d * d) * xw_ref[...] + b_ref[...], 0.0)
    xw2 = jnp.dot(h, w2_ref[...], preferred_element_type=jnp.float32)
    xw2_ref[...] = xw2
    xws2_ref[...] = _pad_rows(xw2 * d, _NR)


def _tcc2_body(sp_ref, xw_ref, dis_ref, b_ref, wm_ref, bm_ref, hm_ref):
    spv = sp_ref[...]
    ssum = spv[0, :_N, :] + spv[1, :_N, :]
    d = dis_ref[...]
    h = jnp.maximum(d * ssum + (d * d) * xw_ref[...] + b_ref[...], 0.0)
    hm = jnp.dot(h, wm_ref[...],
                 preferred_element_type=jnp.float32) + bm_ref[...]
    hm_ref[...] = _pad_rows(hm, _NR)


def _tcd_body(sp_ref, cnt_ref, batch_ref, w3_ref, b3_ref, w4_ref, b4_ref,
              out_ref):
    spv = sp_ref[...]
    ssum = spv[0, :_N, :] + spv[1, :_N, :]
    agg = ssum / jnp.maximum(cnt_ref[...], 1.0)
    aggc = jnp.concatenate([agg, jnp.ones((_N, 1), jnp.float32)], axis=1)
    oh = (lax.broadcasted_iota(jnp.int32, (_N, _NG), 1)
          == batch_ref[...]).astype(jnp.float32)
    gsum = lax.dot_general(oh, aggc, (((0,), (0,)), ((), ())),
                           preferred_element_type=jnp.float32)
    g = gsum[:, :_H] / jnp.maximum(gsum[:, _H:_H + 1], 1.0)
    z = jnp.maximum(
        jnp.dot(g, w3_ref[...], preferred_element_type=jnp.float32)
        + b3_ref[...], 0.0)
    zz = (jnp.dot(z, w4_ref[...], preferred_element_type=jnp.float32)
          + b4_ref[...])
    out_ref[...] = 1.0 / (1.0 + jnp.exp(-zz))


def _sds(shape):
    return jax.ShapeDtypeStruct(shape, jnp.float32)


def kernel(x, edge_index, edge_attr, batch,
           W1, b1, W2, b2, Wm, bm, W3, b3, W4, b4):
    src = edge_index[0]
    dst = edge_index[1]
    attr0 = edge_attr[:, 0]
    pad = _EP - _E
    src3 = jnp.pad(src, (0, pad)).reshape(_NW, _K, _CH)
    dst3 = jnp.pad(dst, (0, pad), constant_values=_N).reshape(_NW, _K, _CH)
    attr3 = jnp.pad(attr0, (0, pad)).reshape(_NW, _K, _CH)
    zeros16 = jnp.zeros((_NR, _DW), jnp.float32)
    zerosH = jnp.zeros((_NR, _H), jnp.float32)
    ones16 = jnp.ones((_CH, _DW), jnp.float32)

    degp = _deg_kernel()(dst3, zeros16, ones16)
    xw1 = pl.pallas_call(_mm_body, out_shape=_sds((_N, _H)))(x, W1)
    dis, cnt, xws1 = pl.pallas_call(
        _tcb_body,
        out_shape=[_sds((_N, 1)), _sds((_N, 1)), _sds((_NR, _H))],
    )(degp, xw1)

    sp1 = _conv_kernel()(xws1, src3, dst3, zerosH)
    xw2, xws2 = pl.pallas_call(
        _tcc1_body,
        out_shape=[_sds((_N, _H)), _sds((_NR, _H))],
    )(sp1, xw1, dis, b1.reshape(1, _H), W2)

    sp2 = _conv_kernel()(xws2, src3, dst3, zerosH)
    hm = pl.pallas_call(
        _tcc2_body,
        out_shape=_sds((_NR, 4 * _H)),
    )(sp2, xw2, dis, b2.reshape(1, _H), Wm, bm.reshape(1, 4 * _H))

    hm_bf = hm.reshape(_NT, _H).astype(jnp.bfloat16)
    table4i = jax.lax.bitcast_convert_type(
        hm_bf.reshape(_NT, 2, _H // 2).transpose(0, 2, 1), jnp.int32)
    sp3 = _mpnn_kernel()(table4i, src3, dst3, attr3, zerosH)

    out = pl.pallas_call(
        _tcd_body,
        out_shape=_sds((_NG, 1)),
    )(sp3, cnt, batch.reshape(_N, 1), W3, b3.reshape(1, _H), W4,
      b4.reshape(1, 1))
    return out


# speedup vs baseline: 1.6971x; 1.6971x over previous
import functools

import jax
import jax.numpy as jnp
from jax import lax
from jax.experimental import pallas as pl
from jax.experimental.pallas import tpu as pltpu
from jax.experimental.pallas import tpu_sc as plsc

_N = 10000
_E = 320000
_D = 128
_H = 32
_NG = 64

_NC = 2
_NS = 16
_NW = _NC * _NS
_CH = 128
_K = 80
_EP = _NW * _K * _CH
_NR = 10112
_RT = _NR // _NS
_TS = _NR // _NS
_NT = 4 * _NR
_DW = 16


def _sc_mesh():
    return plsc.VectorSubcoreMesh(
        core_axis_name="c", subcore_axis_name="s",
        num_cores=_NC, num_subcores=_NS)


@functools.cache
def _deg_kernel():
    @functools.partial(
        pl.kernel,
        out_type=jax.ShapeDtypeStruct((_NC, _NR, _DW), jnp.float32),
        mesh=_sc_mesh(),
        compiler_params=pltpu.CompilerParams(use_tc_tiling_on_sc=False),
        scratch_types=[
            pltpu.VMEM((_K, _CH), jnp.int32),
            pltpu.VMEM((_CH, _DW), jnp.float32),
            pltpu.VMEM_SHARED((_NR, _DW), jnp.float32),
        ],
    )
    def deg(dst3, zeros16, ones16, out, dst_v, ones_v, acc):
        c = lax.axis_index("c")
        s = lax.axis_index("s")
        wid = s * _NC + c
        pltpu.sync_copy(dst3.at[wid], dst_v)
        pltpu.sync_copy(ones16, ones_v)
        pltpu.sync_copy(zeros16.at[pl.ds(s * _RT, _RT)],
                        acc.at[pl.ds(s * _RT, _RT)])
        plsc.subcore_barrier()

        def body(j, carry):
            pltpu.sync_copy(ones_v, acc.at[dst_v.at[j]], add=True)
            return carry

        lax.fori_loop(0, _K, body, 0)
        plsc.subcore_barrier()
        pltpu.sync_copy(acc.at[pl.ds(s * _RT, _RT)],
                        out.at[c, pl.ds(s * _RT, _RT)])

    return deg


@functools.cache
def _conv_kernel():
    nbuf = 8
    hb = nbuf // 2

    @functools.partial(
        pl.kernel,
        out_type=jax.ShapeDtypeStruct((_NC, _NR, _H), jnp.float32),
        mesh=_sc_mesh(),
        compiler_params=pltpu.CompilerParams(use_tc_tiling_on_sc=False),
        scratch_types=[
            pltpu.VMEM((_K, _CH), jnp.int32),
            pltpu.VMEM((_K, _CH), jnp.int32),
            pltpu.VMEM((nbuf, _CH, _H), jnp.float32),
            pltpu.VMEM_SHARED((_NR, _H), jnp.float32),
            pltpu.VMEM_SHARED((_NR, _H), jnp.float32),
            pltpu.SemaphoreType.DMA,
            pltpu.SemaphoreType.DMA,
            pltpu.SemaphoreType.DMA,
            pltpu.SemaphoreType.DMA,
        ],
    )
    def conv(table, src3, dst3, zerosH, out,
             src_v, dst_v, rows_v, table_sp, acc,
             sem_a, sem_b, sem_sa, sem_sb):
        c = lax.axis_index("c")
        s = lax.axis_index("s")
        wid = s * _NC + c
        pltpu.sync_copy(src3.at[wid], src_v)
        pltpu.sync_copy(dst3.at[wid], dst_v)
        pltpu.sync_copy(table.at[pl.ds(s * _TS, _TS)],
                        table_sp.at[pl.ds(s * _TS, _TS)])
        pltpu.sync_copy(zerosH.at[pl.ds(s * _RT, _RT)],
                        acc.at[pl.ds(s * _RT, _RT)])
        plsc.subcore_barrier()

        ng = _K // nbuf

        def fire(cbase, bufbase, sem):
            for b in range(hb):
                pltpu.async_copy(table_sp.at[src_v.at[cbase + b]],
                                 rows_v.at[bufbase + b], sem)

        def wait_gathers(cbase, bufbase, gsem):
            for b in range(hb):
                pltpu.make_async_copy(table_sp.at[src_v.at[cbase + b]],
                                      rows_v.at[bufbase + b], gsem).wait()

        def issue_scatters(cbase, bufbase, ssem):
            for b in range(hb):
                pltpu.async_copy(rows_v.at[bufbase + b],
                                 acc.at[dst_v.at[cbase + b]], ssem,
                                 add=True)

        def drain_scatters(cbase, bufbase, ssem):
            for b in range(hb):
                pltpu.make_async_copy(
                    rows_v.at[bufbase + b],
                    acc.at[dst_v.at[cbase + b]], ssem).wait()

        fire(0, 0, sem_a)

        def group(g, carry):
            base = g * nbuf
            fire(base + hb, hb, sem_b)
            wait_gathers(base, 0, sem_a)
            issue_scatters(base, 0, sem_sa)
            drain_scatters(base, 0, sem_sa)

            @pl.when(g < ng - 1)
            def _():
                fire(base + nbuf, 0, sem_a)

            wait_gathers(base + hb, hb, sem_b)
            issue_scatters(base + hb, hb, sem_sb)
            drain_scatters(base + hb, hb, sem_sb)
            return carry

        lax.fori_loop(0, ng, group, 0)
        plsc.subcore_barrier()
        pltpu.sync_copy(acc.at[pl.ds(s * _RT, _RT)],
                        out.at[c, pl.ds(s * _RT, _RT)])

    return conv


@functools.cache
def _mpnn_kernel():
    nbuf = 16
    hb = nbuf // 2

    @functools.partial(
        pl.kernel,
        out_type=jax.ShapeDtypeStruct((_NC, _NR, _H), jnp.float32),
        mesh=_sc_mesh(),
        compiler_params=pltpu.CompilerParams(use_tc_tiling_on_sc=False),
        scratch_types=[
            pltpu.VMEM((_K, _CH), jnp.int32),
            pltpu.VMEM((_K, _CH), jnp.int32),
            pltpu.VMEM((_K, _CH), jnp.int32),
            pltpu.VMEM((nbuf, _CH), jnp.int32),
            pltpu.VMEM((nbuf, _CH, _H), jnp.float32),
            pltpu.VMEM_SHARED((_NR, _H), jnp.float32),
            pltpu.SemaphoreType.DMA,
            pltpu.SemaphoreType.DMA,
        ],
    )
    def mpnn(table4, src3, dst3, attr3, zerosH, out,
             src_v, dst_v, attr_v, ridx_v, rows_v, acc, sem_a, sem_b):
        c = lax.axis_index("c")
        s = lax.axis_index("s")
        wid = s * _NC + c
        pltpu.sync_copy(src3.at[wid], src_v)
        pltpu.sync_copy(dst3.at[wid], dst_v)
        pltpu.sync_copy(attr3.at[wid], attr_v)
        pltpu.sync_copy(zerosH.at[pl.ds(s * _RT, _RT)],
                        acc.at[pl.ds(s * _RT, _RT)])
        plsc.subcore_barrier()

        ng = _K // nbuf

        def fire(cbase, bufbase, sem):
            for b in range(hb):
                j = cbase + b
                bb = bufbase + b
                for q in range(_CH // 16):
                    sv = src_v[j, pl.ds(q * 16, 16)]
                    av = attr_v[j, pl.ds(q * 16, 16)]
                    ridx_v[bb, pl.ds(q * 16, 16)] = sv * 4 + av
                pltpu.async_copy(table4.at[ridx_v.at[bb]],
                                 rows_v.at[bb], sem)

        def drain_scatter(cbase, bufbase, sem):
            for b in range(hb):
                pltpu.make_async_copy(table4.at[ridx_v.at[bufbase + b]],
                                      rows_v.at[bufbase + b], sem).wait()
            for b in range(hb):
                pltpu.sync_copy(rows_v.at[bufbase + b],
                                acc.at[dst_v.at[cbase + b]], add=True)

        fire(0, 0, sem_a)

        def group(g, carry):
            base = g * nbuf
            fire(base + hb, hb, sem_b)
            drain_scatter(base, 0, sem_a)

            @pl.when(g < ng - 1)
            def _():
                fire(base + nbuf, 0, sem_a)

            drain_scatter(base + hb, hb, sem_b)
            return carry

        lax.fori_loop(0, ng, group, 0)
        plsc.subcore_barrier()
        pltpu.sync_copy(acc.at[pl.ds(s * _RT, _RT)],
                        out.at[c, pl.ds(s * _RT, _RT)])

    return mpnn


def _pad_rows(a, rows):
    return jnp.concatenate(
        [a, jnp.zeros((rows - a.shape[0], a.shape[1]), a.dtype)], axis=0)


def _mm_body(x_ref, w_ref, o_ref):
    o_ref[...] = jnp.dot(x_ref[...], w_ref[...],
                         preferred_element_type=jnp.float32)


def _tcb_body(degp_ref, xw1_ref, dis_ref, cnt_ref, xws_ref):
    dp = degp_ref[...]
    cnt = dp[0, :_N, 0:1] + dp[1, :_N, 0:1]
    dis = lax.rsqrt(cnt + 1.0)
    dis_ref[...] = dis
    cnt_ref[...] = cnt
    xws_ref[...] = _pad_rows(xw1_ref[...] * dis, _NR)


def _tcc1_body(sp_ref, xw_ref, dis_ref, b_ref, w2_ref, xw2_ref, xws2_ref):
    spv = sp_ref[...]
    ssum = spv[0, :_N, :] + spv[1, :_N, :]
    d = dis_ref[...]
    h = jnp.maximum(d * ssum + (d * d) * xw_ref[...] + b_ref[...], 0.0)
    xw2 = jnp.dot(h, w2_ref[...], preferred_element_type=jnp.float32)
    xw2_ref[...] = xw2
    xws2_ref[...] = _pad_rows(xw2 * d, _NR)


def _tcc2_body(sp_ref, xw_ref, dis_ref, b_ref, wm_ref, bm_ref, hm_ref):
    spv = sp_ref[...]
    ssum = spv[0, :_N, :] + spv[1, :_N, :]
    d = dis_ref[...]
    h = jnp.maximum(d * ssum + (d * d) * xw_ref[...] + b_ref[...], 0.0)
    hm = jnp.dot(h, wm_ref[...],
                 preferred_element_type=jnp.float32) + bm_ref[...]
    hm_ref[...] = _pad_rows(hm, _NR)


def _tcd_body(sp_ref, cnt_ref, batch_ref, w3_ref, b3_ref, w4_ref, b4_ref,
              out_ref):
    spv = sp_ref[...]
    ssum = spv[0, :_N, :] + spv[1, :_N, :]
    agg = ssum / jnp.maximum(cnt_ref[...], 1.0)
    aggc = jnp.concatenate([agg, jnp.ones((_N, 1), jnp.float32)], axis=1)
    oh = (lax.broadcasted_iota(jnp.int32, (_N, _NG), 1)
          == batch_ref[...]).astype(jnp.float32)
    gsum = lax.dot_general(oh, aggc, (((0,), (0,)), ((), ())),
                           preferred_element_type=jnp.float32)
    g = gsum[:, :_H] / jnp.maximum(gsum[:, _H:_H + 1], 1.0)
    z = jnp.maximum(
        jnp.dot(g, w3_ref[...], preferred_element_type=jnp.float32)
        + b3_ref[...], 0.0)
    zz = (jnp.dot(z, w4_ref[...], preferred_element_type=jnp.float32)
          + b4_ref[...])
    out_ref[...] = 1.0 / (1.0 + jnp.exp(-zz))


def _sds(shape):
    return jax.ShapeDtypeStruct(shape, jnp.float32)


def kernel(x, edge_index, edge_attr, batch,
           W1, b1, W2, b2, Wm, bm, W3, b3, W4, b4):
    src = edge_index[0]
    dst = edge_index[1]
    attr0 = edge_attr[:, 0]
    pad = _EP - _E
    src3 = jnp.pad(src, (0, pad)).reshape(_NW, _K, _CH)
    dst3 = jnp.pad(dst, (0, pad), constant_values=_N).reshape(_NW, _K, _CH)
    attr3 = jnp.pad(attr0, (0, pad)).reshape(_NW, _K, _CH)
    zeros16 = jnp.zeros((_NR, _DW), jnp.float32)
    zerosH = jnp.zeros((_NR, _H), jnp.float32)
    ones16 = jnp.ones((_CH, _DW), jnp.float32)

    degp = _deg_kernel()(dst3, zeros16, ones16)
    xw1 = pl.pallas_call(_mm_body, out_shape=_sds((_N, _H)))(x, W1)
    dis, cnt, xws1 = pl.pallas_call(
        _tcb_body,
        out_shape=[_sds((_N, 1)), _sds((_N, 1)), _sds((_NR, _H))],
    )(degp, xw1)

    sp1 = _conv_kernel()(xws1, src3, dst3, zerosH)
    xw2, xws2 = pl.pallas_call(
        _tcc1_body,
        out_shape=[_sds((_N, _H)), _sds((_NR, _H))],
    )(sp1, xw1, dis, b1.reshape(1, _H), W2)

    sp2 = _conv_kernel()(xws2, src3, dst3, zerosH)
    hm = pl.pallas_call(
        _tcc2_body,
        out_shape=_sds((_NR, 4 * _H)),
    )(sp2, xw2, dis, b2.reshape(1, _H), Wm, bm.reshape(1, 4 * _H))

    sp3 = _mpnn_kernel()(hm.reshape(_NT, _H), src3, dst3, attr3, zerosH)

    out = pl.pallas_call(
        _tcd_body,
        out_shape=_sds((_NG, 1)),
    )(sp3, cnt, batch.reshape(_N, 1), W3, b3.reshape(1, _H), W4,
      b4.reshape(1, 1))
    return out


# speedup vs baseline: 1.7008x; 1.0022x over previous
import functools

import jax
import jax.numpy as jnp
from jax import lax
from jax.experimental import pallas as pl
from jax.experimental.pallas import tpu as pltpu
from jax.experimental.pallas import tpu_sc as plsc

_N = 10000
_E = 320000
_D = 128
_H = 32
_NG = 64

_NC = 2
_NS = 16
_NW = _NC * _NS
_CH = 128
_K = 80
_EP = _NW * _K * _CH
_NR = 10112
_RT = _NR // _NS
_TS = _NR // _NS
_NT = 4 * _NR
_DW = 16


def _sc_mesh():
    return plsc.VectorSubcoreMesh(
        core_axis_name="c", subcore_axis_name="s",
        num_cores=_NC, num_subcores=_NS)


@functools.cache
def _deg_kernel():
    @functools.partial(
        pl.kernel,
        out_type=jax.ShapeDtypeStruct((_NC, _NR, _DW), jnp.float32),
        mesh=_sc_mesh(),
        compiler_params=pltpu.CompilerParams(use_tc_tiling_on_sc=False),
        scratch_types=[
            pltpu.VMEM((_K, _CH), jnp.int32),
            pltpu.VMEM((_CH, _DW), jnp.float32),
            pltpu.VMEM_SHARED((_NR, _DW), jnp.float32),
        ],
    )
    def deg(dst3, zeros16, ones16, out, dst_v, ones_v, acc):
        c = lax.axis_index("c")
        s = lax.axis_index("s")
        wid = s * _NC + c
        pltpu.sync_copy(dst3.at[wid], dst_v)
        pltpu.sync_copy(ones16, ones_v)
        pltpu.sync_copy(zeros16.at[pl.ds(s * _RT, _RT)],
                        acc.at[pl.ds(s * _RT, _RT)])
        plsc.subcore_barrier()

        def body(j, carry):
            pltpu.sync_copy(ones_v, acc.at[dst_v.at[j]], add=True)
            return carry

        lax.fori_loop(0, _K, body, 0)
        plsc.subcore_barrier()
        pltpu.sync_copy(acc.at[pl.ds(s * _RT, _RT)],
                        out.at[c, pl.ds(s * _RT, _RT)])

    return deg


@functools.cache
def _conv_kernel():
    nbuf = 8
    hb = nbuf // 2

    @functools.partial(
        pl.kernel,
        out_type=jax.ShapeDtypeStruct((_NC, _NR, _H), jnp.float32),
        mesh=_sc_mesh(),
        compiler_params=pltpu.CompilerParams(use_tc_tiling_on_sc=False),
        scratch_types=[
            pltpu.VMEM((_K, _CH), jnp.int32),
            pltpu.VMEM((_K, _CH), jnp.int32),
            pltpu.VMEM((nbuf, _CH, _H), jnp.float32),
            pltpu.VMEM_SHARED((_NR, _H), jnp.float32),
            pltpu.VMEM_SHARED((_NR, _H), jnp.float32),
            pltpu.SemaphoreType.DMA,
            pltpu.SemaphoreType.DMA,
            pltpu.SemaphoreType.DMA,
            pltpu.SemaphoreType.DMA,
        ],
    )
    def conv(table, src3, dst3, zerosH, out,
             src_v, dst_v, rows_v, table_sp, acc,
             sem_a, sem_b, sem_sa, sem_sb):
        c = lax.axis_index("c")
        s = lax.axis_index("s")
        wid = s * _NC + c
        pltpu.sync_copy(src3.at[wid], src_v)
        pltpu.sync_copy(dst3.at[wid], dst_v)
        pltpu.sync_copy(table.at[pl.ds(s * _TS, _TS)],
                        table_sp.at[pl.ds(s * _TS, _TS)])
        pltpu.sync_copy(zerosH.at[pl.ds(s * _RT, _RT)],
                        acc.at[pl.ds(s * _RT, _RT)])
        plsc.subcore_barrier()

        ng = _K // nbuf

        def fire(cbase, bufbase, sem):
            for b in range(hb):
                pltpu.async_copy(table_sp.at[src_v.at[cbase + b]],
                                 rows_v.at[bufbase + b], sem)

        def wait_gathers(cbase, bufbase, gsem):
            for b in range(hb):
                pltpu.make_async_copy(table_sp.at[src_v.at[cbase + b]],
                                      rows_v.at[bufbase + b], gsem).wait()

        def issue_scatters(cbase, bufbase, ssem):
            for b in range(hb):
                pltpu.async_copy(rows_v.at[bufbase + b],
                                 acc.at[dst_v.at[cbase + b]], ssem,
                                 add=True)

        def drain_scatters(cbase, bufbase, ssem):
            for b in range(hb):
                pltpu.make_async_copy(
                    rows_v.at[bufbase + b],
                    acc.at[dst_v.at[cbase + b]], ssem).wait()

        fire(0, 0, sem_a)

        def group(g, carry):
            base = g * nbuf
            fire(base + hb, hb, sem_b)
            wait_gathers(base, 0, sem_a)
            issue_scatters(base, 0, sem_sa)
            drain_scatters(base, 0, sem_sa)

            @pl.when(g < ng - 1)
            def _():
                fire(base + nbuf, 0, sem_a)

            wait_gathers(base + hb, hb, sem_b)
            issue_scatters(base + hb, hb, sem_sb)
            drain_scatters(base + hb, hb, sem_sb)
            return carry

        lax.fori_loop(0, ng, group, 0)
        plsc.subcore_barrier()
        pltpu.sync_copy(acc.at[pl.ds(s * _RT, _RT)],
                        out.at[c, pl.ds(s * _RT, _RT)])

    return conv


@functools.cache
def _mpnn_kernel():
    nbuf = 16
    hb = nbuf // 2

    @functools.partial(
        pl.kernel,
        out_type=jax.ShapeDtypeStruct((_NC, _NR, _H), jnp.float32),
        mesh=_sc_mesh(),
        compiler_params=pltpu.CompilerParams(use_tc_tiling_on_sc=False),
        scratch_types=[
            pltpu.VMEM((_K, _CH), jnp.int32),
            pltpu.VMEM((_K, _CH), jnp.int32),
            pltpu.VMEM((_K, _CH), jnp.int32),
            pltpu.VMEM((nbuf, _CH), jnp.int32),
            pltpu.VMEM((nbuf, _CH, _H), jnp.float32),
            pltpu.VMEM_SHARED((_NR, _H), jnp.float32),
            pltpu.SemaphoreType.DMA,
            pltpu.SemaphoreType.DMA,
            pltpu.SemaphoreType.DMA,
            pltpu.SemaphoreType.DMA,
        ],
    )
    def mpnn(table4, src3, dst3, attr3, zerosH, out,
             src_v, dst_v, attr_v, ridx_v, rows_v, acc,
             sem_a, sem_b, sem_sa, sem_sb):
        c = lax.axis_index("c")
        s = lax.axis_index("s")
        wid = s * _NC + c
        pltpu.sync_copy(src3.at[wid], src_v)
        pltpu.sync_copy(dst3.at[wid], dst_v)
        pltpu.sync_copy(attr3.at[wid], attr_v)
        pltpu.sync_copy(zerosH.at[pl.ds(s * _RT, _RT)],
                        acc.at[pl.ds(s * _RT, _RT)])
        plsc.subcore_barrier()

        ng = _K // nbuf

        def fire(cbase, bufbase, sem):
            for b in range(hb):
                j = cbase + b
                bb = bufbase + b
                for q in range(_CH // 16):
                    sv = src_v[j, pl.ds(q * 16, 16)]
                    av = attr_v[j, pl.ds(q * 16, 16)]
                    ridx_v[bb, pl.ds(q * 16, 16)] = sv * 4 + av
                pltpu.async_copy(table4.at[ridx_v.at[bb]],
                                 rows_v.at[bb], sem)

        def drain_scatter(cbase, bufbase, sem, ssem):
            for b in range(hb):
                pltpu.make_async_copy(table4.at[ridx_v.at[bufbase + b]],
                                      rows_v.at[bufbase + b], sem).wait()
            for b in range(hb):
                pltpu.async_copy(rows_v.at[bufbase + b],
                                 acc.at[dst_v.at[cbase + b]], ssem,
                                 add=True)
            for b in range(hb):
                pltpu.make_async_copy(
                    rows_v.at[bufbase + b],
                    acc.at[dst_v.at[cbase + b]], ssem).wait()

        fire(0, 0, sem_a)

        def group(g, carry):
            base = g * nbuf
            fire(base + hb, hb, sem_b)
            drain_scatter(base, 0, sem_a, sem_sa)

            @pl.when(g < ng - 1)
            def _():
                fire(base + nbuf, 0, sem_a)

            drain_scatter(base + hb, hb, sem_b, sem_sb)
            return carry

        lax.fori_loop(0, ng, group, 0)
        plsc.subcore_barrier()
        pltpu.sync_copy(acc.at[pl.ds(s * _RT, _RT)],
                        out.at[c, pl.ds(s * _RT, _RT)])

    return mpnn


def _pad_rows(a, rows):
    return jnp.concatenate(
        [a, jnp.zeros((rows - a.shape[0], a.shape[1]), a.dtype)], axis=0)


def _mm_body(x_ref, w_ref, o_ref):
    o_ref[...] = jnp.dot(x_ref[...], w_ref[...],
                         preferred_element_type=jnp.float32)


def _tcb_body(degp_ref, xw1_ref, dis_ref, cnt_ref, xws_ref):
    dp = degp_ref[...]
    cnt = dp[0, :_N, 0:1] + dp[1, :_N, 0:1]
    dis = lax.rsqrt(cnt + 1.0)
    dis_ref[...] = dis
    cnt_ref[...] = cnt
    xws_ref[...] = _pad_rows(xw1_ref[...] * dis, _NR)


def _tcc1_body(sp_ref, xw_ref, dis_ref, b_ref, w2_ref, xw2_ref, xws2_ref):
    spv = sp_ref[...]
    ssum = spv[0, :_N, :] + spv[1, :_N, :]
    d = dis_ref[...]
    h = jnp.maximum(d * ssum + (d * d) * xw_ref[...] + b_ref[...], 0.0)
    xw2 = jnp.dot(h, w2_ref[...], preferred_element_type=jnp.float32)
    xw2_ref[...] = xw2
    xws2_ref[...] = _pad_rows(xw2 * d, _NR)


def _tcc2_body(sp_ref, xw_ref, dis_ref, b_ref, wm_ref, bm_ref, hm_ref):
    spv = sp_ref[...]
    ssum = spv[0, :_N, :] + spv[1, :_N, :]
    d = dis_ref[...]
    h = jnp.maximum(d * ssum + (d * d) * xw_ref[...] + b_ref[...], 0.0)
    hm = jnp.dot(h, wm_ref[...],
                 preferred_element_type=jnp.float32) + bm_ref[...]
    hm_ref[...] = _pad_rows(hm, _NR)


def _tcd_body(sp_ref, cnt_ref, batch_ref, w3_ref, b3_ref, w4_ref, b4_ref,
              out_ref):
    spv = sp_ref[...]
    ssum = spv[0, :_N, :] + spv[1, :_N, :]
    agg = ssum / jnp.maximum(cnt_ref[...], 1.0)
    aggc = jnp.concatenate([agg, jnp.ones((_N, 1), jnp.float32)], axis=1)
    oh = (lax.broadcasted_iota(jnp.int32, (_N, _NG), 1)
          == batch_ref[...]).astype(jnp.float32)
    gsum = lax.dot_general(oh, aggc, (((0,), (0,)), ((), ())),
                           preferred_element_type=jnp.float32)
    g = gsum[:, :_H] / jnp.maximum(gsum[:, _H:_H + 1], 1.0)
    z = jnp.maximum(
        jnp.dot(g, w3_ref[...], preferred_element_type=jnp.float32)
        + b3_ref[...], 0.0)
    zz = (jnp.dot(z, w4_ref[...], preferred_element_type=jnp.float32)
          + b4_ref[...])
    out_ref[...] = 1.0 / (1.0 + jnp.exp(-zz))


def _sds(shape):
    return jax.ShapeDtypeStruct(shape, jnp.float32)


def kernel(x, edge_index, edge_attr, batch,
           W1, b1, W2, b2, Wm, bm, W3, b3, W4, b4):
    src = edge_index[0]
    dst = edge_index[1]
    attr0 = edge_attr[:, 0]
    pad = _EP - _E
    src3 = jnp.pad(src, (0, pad)).reshape(_NW, _K, _CH)
    dst3 = jnp.pad(dst, (0, pad), constant_values=_N).reshape(_NW, _K, _CH)
    attr3 = jnp.pad(attr0, (0, pad)).reshape(_NW, _K, _CH)
    zeros16 = jnp.zeros((_NR, _DW), jnp.float32)
    zerosH = jnp.zeros((_NR, _H), jnp.float32)
    ones16 = jnp.ones((_CH, _DW), jnp.float32)

    degp = _deg_kernel()(dst3, zeros16, ones16)
    xw1 = pl.pallas_call(_mm_body, out_shape=_sds((_N, _H)))(x, W1)
    dis, cnt, xws1 = pl.pallas_call(
        _tcb_body,
        out_shape=[_sds((_N, 1)), _sds((_N, 1)), _sds((_NR, _H))],
    )(degp, xw1)

    sp1 = _conv_kernel()(xws1, src3, dst3, zerosH)
    xw2, xws2 = pl.pallas_call(
        _tcc1_body,
        out_shape=[_sds((_N, _H)), _sds((_NR, _H))],
    )(sp1, xw1, dis, b1.reshape(1, _H), W2)

    sp2 = _conv_kernel()(xws2, src3, dst3, zerosH)
    hm = pl.pallas_call(
        _tcc2_body,
        out_shape=_sds((_NR, 4 * _H)),
    )(sp2, xw2, dis, b2.reshape(1, _H), Wm, bm.reshape(1, 4 * _H))

    sp3 = _mpnn_kernel()(hm.reshape(_NT, _H), src3, dst3, attr3, zerosH)

    out = pl.pallas_call(
        _tcd_body,
        out_shape=_sds((_NG, 1)),
    )(sp3, cnt, batch.reshape(_N, 1), W3, b3.reshape(1, _H), W4,
      b4.reshape(1, 1))
    return out


# speedup vs baseline: 1.7013x; 1.0003x over previous
import functools

import jax
import jax.numpy as jnp
from jax import lax
from jax.experimental import pallas as pl
from jax.experimental.pallas import tpu as pltpu
from jax.experimental.pallas import tpu_sc as plsc

_N = 10000
_E = 320000
_D = 128
_H = 32
_NG = 64

_NC = 2
_NS = 16
_NW = _NC * _NS
_CH = 128
_K = 80
_EP = _NW * _K * _CH
_NR = 10112
_RT = _NR // _NS
_TS = _NR // _NS
_NT = 4 * _NR
_DW = 16


def _sc_mesh():
    return plsc.VectorSubcoreMesh(
        core_axis_name="c", subcore_axis_name="s",
        num_cores=_NC, num_subcores=_NS)


@functools.cache
def _deg_kernel():
    @functools.partial(
        pl.kernel,
        out_type=jax.ShapeDtypeStruct((_NC, _NR, _DW), jnp.float32),
        mesh=_sc_mesh(),
        compiler_params=pltpu.CompilerParams(use_tc_tiling_on_sc=False),
        scratch_types=[
            pltpu.VMEM((_K, _CH), jnp.int32),
            pltpu.VMEM((_CH, _DW), jnp.float32),
            pltpu.VMEM_SHARED((_NR, _DW), jnp.float32),
        ],
    )
    def deg(dst3, zeros16, ones16, out, dst_v, ones_v, acc):
        c = lax.axis_index("c")
        s = lax.axis_index("s")
        wid = s * _NC + c
        pltpu.sync_copy(dst3.at[wid], dst_v)
        pltpu.sync_copy(ones16, ones_v)
        pltpu.sync_copy(zeros16.at[pl.ds(s * _RT, _RT)],
                        acc.at[pl.ds(s * _RT, _RT)])
        plsc.subcore_barrier()

        def body(j, carry):
            pltpu.sync_copy(ones_v, acc.at[dst_v.at[j]], add=True)
            return carry

        lax.fori_loop(0, _K, body, 0)
        plsc.subcore_barrier()
        pltpu.sync_copy(acc.at[pl.ds(s * _RT, _RT)],
                        out.at[c, pl.ds(s * _RT, _RT)])

    return deg


@functools.cache
def _conv_kernel():
    nbuf = 16
    hb = nbuf // 2

    @functools.partial(
        pl.kernel,
        out_type=jax.ShapeDtypeStruct((_NC, _NR, _H), jnp.float32),
        mesh=_sc_mesh(),
        compiler_params=pltpu.CompilerParams(use_tc_tiling_on_sc=False),
        scratch_types=[
            pltpu.VMEM((_K, _CH), jnp.int32),
            pltpu.VMEM((_K, _CH), jnp.int32),
            pltpu.VMEM((nbuf, _CH, _H), jnp.float32),
            pltpu.VMEM_SHARED((_NR, _H), jnp.float32),
            pltpu.VMEM_SHARED((_NR, _H), jnp.float32),
            pltpu.SemaphoreType.DMA,
            pltpu.SemaphoreType.DMA,
            pltpu.SemaphoreType.DMA,
            pltpu.SemaphoreType.DMA,
        ],
    )
    def conv(table, src3, dst3, zerosH, out,
             src_v, dst_v, rows_v, table_sp, acc,
             sem_a, sem_b, sem_sa, sem_sb):
        c = lax.axis_index("c")
        s = lax.axis_index("s")
        wid = s * _NC + c
        pltpu.sync_copy(src3.at[wid], src_v)
        pltpu.sync_copy(dst3.at[wid], dst_v)
        pltpu.sync_copy(table.at[pl.ds(s * _TS, _TS)],
                        table_sp.at[pl.ds(s * _TS, _TS)])
        pltpu.sync_copy(zerosH.at[pl.ds(s * _RT, _RT)],
                        acc.at[pl.ds(s * _RT, _RT)])
        plsc.subcore_barrier()

        ng = _K // nbuf

        def fire(cbase, bufbase, sem):
            for b in range(hb):
                pltpu.async_copy(table_sp.at[src_v.at[cbase + b]],
                                 rows_v.at[bufbase + b], sem)

        def wait_gathers(cbase, bufbase, gsem):
            for b in range(hb):
                pltpu.make_async_copy(table_sp.at[src_v.at[cbase + b]],
                                      rows_v.at[bufbase + b], gsem).wait()

        def issue_scatters(cbase, bufbase, ssem):
            for b in range(hb):
                pltpu.async_copy(rows_v.at[bufbase + b],
                                 acc.at[dst_v.at[cbase + b]], ssem,
                                 add=True)

        def drain_scatters(cbase, bufbase, ssem):
            for b in range(hb):
                pltpu.make_async_copy(
                    rows_v.at[bufbase + b],
                    acc.at[dst_v.at[cbase + b]], ssem).wait()

        fire(0, 0, sem_a)

        def group(g, carry):
            base = g * nbuf
            fire(base + hb, hb, sem_b)
            wait_gathers(base, 0, sem_a)
            issue_scatters(base, 0, sem_sa)
            drain_scatters(base, 0, sem_sa)

            @pl.when(g < ng - 1)
            def _():
                fire(base + nbuf, 0, sem_a)

            wait_gathers(base + hb, hb, sem_b)
            issue_scatters(base + hb, hb, sem_sb)
            drain_scatters(base + hb, hb, sem_sb)
            return carry

        lax.fori_loop(0, ng, group, 0)
        plsc.subcore_barrier()
        pltpu.sync_copy(acc.at[pl.ds(s * _RT, _RT)],
                        out.at[c, pl.ds(s * _RT, _RT)])

    return conv


@functools.cache
def _mpnn_kernel():
    nbuf = 16
    hb = nbuf // 2

    @functools.partial(
        pl.kernel,
        out_type=jax.ShapeDtypeStruct((_NC, _NR, _H), jnp.float32),
        mesh=_sc_mesh(),
        compiler_params=pltpu.CompilerParams(use_tc_tiling_on_sc=False),
        scratch_types=[
            pltpu.VMEM((_K, _CH), jnp.int32),
            pltpu.VMEM((_K, _CH), jnp.int32),
            pltpu.VMEM((_K, _CH), jnp.int32),
            pltpu.VMEM((nbuf, _CH), jnp.int32),
            pltpu.VMEM((nbuf, _CH, _H), jnp.float32),
            pltpu.VMEM_SHARED((_NR, _H), jnp.float32),
            pltpu.SemaphoreType.DMA,
            pltpu.SemaphoreType.DMA,
            pltpu.SemaphoreType.DMA,
            pltpu.SemaphoreType.DMA,
        ],
    )
    def mpnn(table4, src3, dst3, attr3, zerosH, out,
             src_v, dst_v, attr_v, ridx_v, rows_v, acc,
             sem_a, sem_b, sem_sa, sem_sb):
        c = lax.axis_index("c")
        s = lax.axis_index("s")
        wid = s * _NC + c
        pltpu.sync_copy(src3.at[wid], src_v)
        pltpu.sync_copy(dst3.at[wid], dst_v)
        pltpu.sync_copy(attr3.at[wid], attr_v)
        pltpu.sync_copy(zerosH.at[pl.ds(s * _RT, _RT)],
                        acc.at[pl.ds(s * _RT, _RT)])
        plsc.subcore_barrier()

        ng = _K // nbuf

        def fire(cbase, bufbase, sem):
            for b in range(hb):
                j = cbase + b
                bb = bufbase + b
                for q in range(_CH // 16):
                    sv = src_v[j, pl.ds(q * 16, 16)]
                    av = attr_v[j, pl.ds(q * 16, 16)]
                    ridx_v[bb, pl.ds(q * 16, 16)] = sv * 4 + av
                pltpu.async_copy(table4.at[ridx_v.at[bb]],
                                 rows_v.at[bb], sem)

        def drain_scatter(cbase, bufbase, sem, ssem):
            for b in range(hb):
                pltpu.make_async_copy(table4.at[ridx_v.at[bufbase + b]],
                                      rows_v.at[bufbase + b], sem).wait()
            for b in range(hb):
                pltpu.async_copy(rows_v.at[bufbase + b],
                                 acc.at[dst_v.at[cbase + b]], ssem,
                                 add=True)
            for b in range(hb):
                pltpu.make_async_copy(
                    rows_v.at[bufbase + b],
                    acc.at[dst_v.at[cbase + b]], ssem).wait()

        fire(0, 0, sem_a)

        def group(g, carry):
            base = g * nbuf
            fire(base + hb, hb, sem_b)
            drain_scatter(base, 0, sem_a, sem_sa)

            @pl.when(g < ng - 1)
            def _():
                fire(base + nbuf, 0, sem_a)

            drain_scatter(base + hb, hb, sem_b, sem_sb)
            return carry

        lax.fori_loop(0, ng, group, 0)
        plsc.subcore_barrier()
        pltpu.sync_copy(acc.at[pl.ds(s * _RT, _RT)],
                        out.at[c, pl.ds(s * _RT, _RT)])

    return mpnn


def _pad_rows(a, rows):
    return jnp.concatenate(
        [a, jnp.zeros((rows - a.shape[0], a.shape[1]), a.dtype)], axis=0)


def _mm_body(x_ref, w_ref, o_ref):
    o_ref[...] = jnp.dot(x_ref[...], w_ref[...],
                         preferred_element_type=jnp.float32)


def _tcb_body(degp_ref, xw1_ref, dis_ref, cnt_ref, xws_ref):
    dp = degp_ref[...]
    cnt = dp[0, :_N, 0:1] + dp[1, :_N, 0:1]
    dis = lax.rsqrt(cnt + 1.0)
    dis_ref[...] = dis
    cnt_ref[...] = cnt
    xws_ref[...] = _pad_rows(xw1_ref[...] * dis, _NR)


def _tcc1_body(sp_ref, xw_ref, dis_ref, b_ref, w2_ref, xw2_ref, xws2_ref):
    spv = sp_ref[...]
    ssum = spv[0, :_N, :] + spv[1, :_N, :]
    d = dis_ref[...]
    h = jnp.maximum(d * ssum + (d * d) * xw_ref[...] + b_ref[...], 0.0)
    xw2 = jnp.dot(h, w2_ref[...], preferred_element_type=jnp.float32)
    xw2_ref[...] = xw2
    xws2_ref[...] = _pad_rows(xw2 * d, _NR)


def _tcc2_body(sp_ref, xw_ref, dis_ref, b_ref, wm_ref, bm_ref, hm_ref):
    spv = sp_ref[...]
    ssum = spv[0, :_N, :] + spv[1, :_N, :]
    d = dis_ref[...]
    h = jnp.maximum(d * ssum + (d * d) * xw_ref[...] + b_ref[...], 0.0)
    hm = jnp.dot(h, wm_ref[...],
                 preferred_element_type=jnp.float32) + bm_ref[...]
    hm_ref[...] = _pad_rows(hm, _NR)


def _tcd_body(sp_ref, cnt_ref, batch_ref, w3_ref, b3_ref, w4_ref, b4_ref,
              out_ref):
    spv = sp_ref[...]
    ssum = spv[0, :_N, :] + spv[1, :_N, :]
    agg = ssum / jnp.maximum(cnt_ref[...], 1.0)
    aggc = jnp.concatenate([agg, jnp.ones((_N, 1), jnp.float32)], axis=1)
    oh = (lax.broadcasted_iota(jnp.int32, (_N, _NG), 1)
          == batch_ref[...]).astype(jnp.float32)
    gsum = lax.dot_general(oh, aggc, (((0,), (0,)), ((), ())),
                           preferred_element_type=jnp.float32)
    g = gsum[:, :_H] / jnp.maximum(gsum[:, _H:_H + 1], 1.0)
    z = jnp.maximum(
        jnp.dot(g, w3_ref[...], preferred_element_type=jnp.float32)
        + b3_ref[...], 0.0)
    zz = (jnp.dot(z, w4_ref[...], preferred_element_type=jnp.float32)
          + b4_ref[...])
    out_ref[...] = 1.0 / (1.0 + jnp.exp(-zz))


def _sds(shape):
    return jax.ShapeDtypeStruct(shape, jnp.float32)


def kernel(x, edge_index, edge_attr, batch,
           W1, b1, W2, b2, Wm, bm, W3, b3, W4, b4):
    src = edge_index[0]
    dst = edge_index[1]
    attr0 = edge_attr[:, 0]
    pad = _EP - _E
    src3 = jnp.pad(src, (0, pad)).reshape(_NW, _K, _CH)
    dst3 = jnp.pad(dst, (0, pad), constant_values=_N).reshape(_NW, _K, _CH)
    attr3 = jnp.pad(attr0, (0, pad)).reshape(_NW, _K, _CH)
    zeros16 = jnp.zeros((_NR, _DW), jnp.float32)
    zerosH = jnp.zeros((_NR, _H), jnp.float32)
    ones16 = jnp.ones((_CH, _DW), jnp.float32)

    degp = _deg_kernel()(dst3, zeros16, ones16)
    xw1 = pl.pallas_call(_mm_body, out_shape=_sds((_N, _H)))(x, W1)
    dis, cnt, xws1 = pl.pallas_call(
        _tcb_body,
        out_shape=[_sds((_N, 1)), _sds((_N, 1)), _sds((_NR, _H))],
    )(degp, xw1)

    sp1 = _conv_kernel()(xws1, src3, dst3, zerosH)
    xw2, xws2 = pl.pallas_call(
        _tcc1_body,
        out_shape=[_sds((_N, _H)), _sds((_NR, _H))],
    )(sp1, xw1, dis, b1.reshape(1, _H), W2)

    sp2 = _conv_kernel()(xws2, src3, dst3, zerosH)
    hm = pl.pallas_call(
        _tcc2_body,
        out_shape=_sds((_NR, 4 * _H)),
    )(sp2, xw2, dis, b2.reshape(1, _H), Wm, bm.reshape(1, 4 * _H))

    sp3 = _mpnn_kernel()(hm.reshape(_NT, _H), src3, dst3, attr3, zerosH)

    out = pl.pallas_call(
        _tcd_body,
        out_shape=_sds((_NG, 1)),
    )(sp3, cnt, batch.reshape(_N, 1), W3, b3.reshape(1, _H), W4,
      b4.reshape(1, 1))
    return out
